# baseline (speedup 1.0000x reference)
import functools

import jax
import jax.numpy as jnp
from jax import lax
from jax.experimental import pallas as pl
from jax.experimental.pallas import tpu as pltpu
from jax.experimental.pallas import tpu_sc as plsc

TOTAL = 32768
D = 256
NSEG = 16
NC = 2
NS = 16
NW = NC * NS
ROWS_PER_W = TOTAL // NW
CHUNK = 128
NCHUNK = ROWS_PER_W // CHUNK
LANES = 16
NVEC = D // LANES


def _sc_segsum_body(
    flat_hbm, cu_hbm, psum_hbm, cu_v, buf0, buf1, acc_v, sem0, sem1
):
  cid = lax.axis_index("c")
  sid = lax.axis_index("s")
  wid = sid * NC + cid
  base = wid * ROWS_PER_W

  pltpu.sync_copy(cu_hbm, cu_v)
  va = cu_v[pl.ds(0, LANES)]
  vb = cu_v[pl.ds(LANES, LANES)]
  cu = [va[i] for i in range(LANES)] + [vb[0]]

  zeros = jnp.zeros((LANES,), jnp.float32)
  for s in range(NSEG):
    for j in range(NVEC):
      acc_v[s, pl.ds(j * LANES, LANES)] = zeros

  bufs = (buf0, buf1)
  sems = (sem0, sem1)

  pltpu.async_copy(flat_hbm.at[pl.ds(base, CHUNK)], bufs[0], sems[0])

  def compute(buf, row0):
    for s in range(NSEG):
      lo = jnp.maximum(cu[s], row0) - row0
      hi = jnp.minimum(cu[s + 1], row0 + CHUNK) - row0
      full = jnp.logical_and(lo == 0, hi == CHUNK)

      @pl.when(full)
      def _():
        init = tuple(acc_v[s, pl.ds(j * LANES, LANES)] for j in range(NVEC))

        @pl.loop(0, CHUNK, init_carry=init, unroll=4)
        def acc(r, carry):
          return tuple(
              carry[j] + buf[r, pl.ds(j * LANES, LANES)]
              for j in range(NVEC)
          )

        for j in range(NVEC):
          acc_v[s, pl.ds(j * LANES, LANES)] = acc[j]

      @pl.when(jnp.logical_and(jnp.logical_not(full), hi > lo))
      def _():
        init = tuple(acc_v[s, pl.ds(j * LANES, LANES)] for j in range(NVEC))

        @pl.loop(lo, hi, init_carry=init)
        def acc(r, carry):
          return tuple(
              carry[j] + buf[r, pl.ds(j * LANES, LANES)]
              for j in range(NVEC)
          )

        for j in range(NVEC):
          acc_v[s, pl.ds(j * LANES, LANES)] = acc[j]

  @pl.loop(0, NCHUNK, step=2)
  def _pair(ch):
    for b in range(2):
      cur = ch + b

      @pl.when(cur + 1 < NCHUNK)
      def _():
        pltpu.async_copy(
            flat_hbm.at[pl.ds(base + (cur + 1) * CHUNK, CHUNK)],
            bufs[1 - b],
            sems[1 - b],
        )

      pltpu.make_async_copy(
          flat_hbm.at[pl.ds(0, CHUNK)], bufs[b], sems[b]
      ).wait()
      compute(bufs[b], base + cur * CHUNK)

  pltpu.sync_copy(acc_v, psum_hbm.at[wid])


@functools.partial(
    pl.kernel,
    out_type=jax.ShapeDtypeStruct((NW, NSEG, D), jnp.float32),
    mesh=plsc.VectorSubcoreMesh(core_axis_name="c", subcore_axis_name="s"),
    scratch_types=[
        pltpu.VMEM((2 * LANES,), jnp.int32),
        pltpu.VMEM((CHUNK, D), jnp.float32),
        pltpu.VMEM((CHUNK, D), jnp.float32),
        pltpu.VMEM((NSEG, D), jnp.float32),
        pltpu.SemaphoreType.DMA,
        pltpu.SemaphoreType.DMA,
    ],
)
def _sc_segsum(flat_hbm, cu_hbm, psum_hbm, cu_v, buf0, buf1, acc_v, s0, s1):
  _sc_segsum_body(flat_hbm, cu_hbm, psum_hbm, cu_v, buf0, buf1, acc_v, s0, s1)


def _mlp_body(cu_ref, psum_ref, w1_ref, b1_ref, w2_ref, b2_ref, out_ref):
  sums = jnp.sum(psum_ref[...], axis=0)
  scales = []
  for s in range(NSEG):
    cnt = (cu_ref[s + 1] - cu_ref[s]).astype(jnp.float32)
    scales.append(jnp.full((1, D), 1.0 / jnp.maximum(cnt, 1.0), jnp.float32))
  mean = sums * jnp.concatenate(scales, axis=0)
  h = jnp.maximum(
      jnp.dot(
          mean,
          w1_ref[...],
          preferred_element_type=jnp.float32,
          precision=lax.Precision.HIGHEST,
      )
      + b1_ref[...],
      0.0,
  )
  out_ref[...] = (
      jnp.dot(
          h,
          w2_ref[...],
          preferred_element_type=jnp.float32,
          precision=lax.Precision.HIGHEST,
      )
      + b2_ref[...]
  )


_mlp_call = pl.pallas_call(
    _mlp_body,
    out_shape=jax.ShapeDtypeStruct((NSEG, D), jnp.float32),
    in_specs=[
        pl.BlockSpec(memory_space=pltpu.SMEM),
        pl.BlockSpec(memory_space=pltpu.VMEM),
        pl.BlockSpec(memory_space=pltpu.VMEM),
        pl.BlockSpec(memory_space=pltpu.VMEM),
        pl.BlockSpec(memory_space=pltpu.VMEM),
        pl.BlockSpec(memory_space=pltpu.VMEM),
    ],
    out_specs=pl.BlockSpec(memory_space=pltpu.VMEM),
)


@jax.jit
def kernel(flat, cu_seqlens, W1, b1, W2, b2):
  cu_pad = jnp.concatenate(
      [cu_seqlens, jnp.full((2 * LANES - NSEG - 1,), TOTAL, jnp.int32)]
  )
  psum = _sc_segsum(flat, cu_pad)
  return _mlp_call(
      cu_seqlens, psum, W1, b1.reshape(1, -1), W2, b2.reshape(1, -1)
  )


# speedup vs baseline: 1.1391x; 1.1391x over previous
import functools

import jax
import jax.numpy as jnp
from jax import lax
from jax.experimental import pallas as pl
from jax.experimental.pallas import tpu as pltpu
from jax.experimental.pallas import tpu_sc as plsc

TOTAL = 32768
D = 256
NSEG = 16
NC = 2
NS = 16
NW = NC * NS
ROWS_PER_W = TOTAL // NW
CHUNK = 128
NCHUNK = ROWS_PER_W // CHUNK
LANES = 16
NVEC = D // LANES


def _sc_segsum_body(
    flat_hbm, cu_hbm, psum_hbm, cu_v, buf0, buf1, acc_v, sem0, sem1
):
  cid = lax.axis_index("c")
  sid = lax.axis_index("s")
  wid = sid * NC + cid
  base = wid * ROWS_PER_W

  pltpu.sync_copy(cu_hbm, cu_v)
  va = cu_v[pl.ds(0, LANES)]
  vb = cu_v[pl.ds(LANES, LANES)]
  cu = [va[i] for i in range(LANES)] + [vb[0]]

  zeros = jnp.zeros((LANES,), jnp.float32)
  for s in range(NSEG):
    for j in range(NVEC):
      acc_v[s, pl.ds(j * LANES, LANES)] = zeros

  bufs = (buf0, buf1)
  sems = (sem0, sem1)

  pltpu.async_copy(flat_hbm.at[pl.ds(base, CHUNK)], bufs[0], sems[0])

  def compute(buf, row0):
    for s in range(NSEG):
      lo = jnp.maximum(cu[s], row0) - row0
      hi = jnp.minimum(cu[s + 1], row0 + CHUNK) - row0
      full = jnp.logical_and(lo == 0, hi == CHUNK)

      @pl.when(full)
      def _():
        init = tuple(acc_v[s, pl.ds(j * LANES, LANES)] for j in range(NVEC))

        @plsc.parallel_loop(0, CHUNK, carry=init, unroll=2)
        def acc(r, carry):
          return tuple(
              carry[j] + buf[r, pl.ds(j * LANES, LANES)]
              for j in range(NVEC)
          )

        for j in range(NVEC):
          acc_v[s, pl.ds(j * LANES, LANES)] = acc[j]

      @pl.when(jnp.logical_and(jnp.logical_not(full), hi > lo))
      def _():
        init = tuple(acc_v[s, pl.ds(j * LANES, LANES)] for j in range(NVEC))

        @pl.loop(lo, hi, init_carry=init)
        def acc(r, carry):
          return tuple(
              carry[j] + buf[r, pl.ds(j * LANES, LANES)]
              for j in range(NVEC)
          )

        for j in range(NVEC):
          acc_v[s, pl.ds(j * LANES, LANES)] = acc[j]

  @pl.loop(0, NCHUNK, step=2)
  def _pair(ch):
    for b in range(2):
      cur = ch + b

      @pl.when(cur + 1 < NCHUNK)
      def _():
        pltpu.async_copy(
            flat_hbm.at[pl.ds(base + (cur + 1) * CHUNK, CHUNK)],
            bufs[1 - b],
            sems[1 - b],
        )

      pltpu.make_async_copy(
          flat_hbm.at[pl.ds(0, CHUNK)], bufs[b], sems[b]
      ).wait()
      compute(bufs[b], base + cur * CHUNK)

  pltpu.sync_copy(acc_v, psum_hbm.at[wid])


@functools.partial(
    pl.kernel,
    out_type=jax.ShapeDtypeStruct((NW, NSEG, D), jnp.float32),
    mesh=plsc.VectorSubcoreMesh(core_axis_name="c", subcore_axis_name="s"),
    scratch_types=[
        pltpu.VMEM((2 * LANES,), jnp.int32),
        pltpu.VMEM((CHUNK, D), jnp.float32),
        pltpu.VMEM((CHUNK, D), jnp.float32),
        pltpu.VMEM((NSEG, D), jnp.float32),
        pltpu.SemaphoreType.DMA,
        pltpu.SemaphoreType.DMA,
    ],
)
def _sc_segsum(flat_hbm, cu_hbm, psum_hbm, cu_v, buf0, buf1, acc_v, s0, s1):
  _sc_segsum_body(flat_hbm, cu_hbm, psum_hbm, cu_v, buf0, buf1, acc_v, s0, s1)


def _mlp_body(cu_ref, psum_ref, w1_ref, b1_ref, w2_ref, b2_ref, out_ref):
  sums = jnp.sum(psum_ref[...], axis=0)
  scales = []
  for s in range(NSEG):
    cnt = (cu_ref[s + 1] - cu_ref[s]).astype(jnp.float32)
    scales.append(jnp.full((1, D), 1.0 / jnp.maximum(cnt, 1.0), jnp.float32))
  mean = sums * jnp.concatenate(scales, axis=0)
  h = jnp.maximum(
      jnp.dot(
          mean,
          w1_ref[...],
          preferred_element_type=jnp.float32,
          precision=lax.Precision.HIGHEST,
      )
      + b1_ref[...],
      0.0,
  )
  out_ref[...] = (
      jnp.dot(
          h,
          w2_ref[...],
          preferred_element_type=jnp.float32,
          precision=lax.Precision.HIGHEST,
      )
      + b2_ref[...]
  )


_mlp_call = pl.pallas_call(
    _mlp_body,
    out_shape=jax.ShapeDtypeStruct((NSEG, D), jnp.float32),
    in_specs=[
        pl.BlockSpec(memory_space=pltpu.SMEM),
        pl.BlockSpec(memory_space=pltpu.VMEM),
        pl.BlockSpec(memory_space=pltpu.VMEM),
        pl.BlockSpec(memory_space=pltpu.VMEM),
        pl.BlockSpec(memory_space=pltpu.VMEM),
        pl.BlockSpec(memory_space=pltpu.VMEM),
    ],
    out_specs=pl.BlockSpec(memory_space=pltpu.VMEM),
)


@jax.jit
def kernel(flat, cu_seqlens, W1, b1, W2, b2):
  cu_pad = jnp.concatenate(
      [cu_seqlens, jnp.full((2 * LANES - NSEG - 1,), TOTAL, jnp.int32)]
  )
  psum = _sc_segsum(flat, cu_pad)
  return _mlp_call(
      cu_seqlens, psum, W1, b1.reshape(1, -1), W2, b2.reshape(1, -1)
  )


# speedup vs baseline: 1.9168x; 1.6828x over previous
import functools

import jax
import jax.numpy as jnp
from jax import lax
from jax.experimental import pallas as pl
from jax.experimental.pallas import tpu as pltpu
from jax.experimental.pallas import tpu_sc as plsc

TOTAL = 32768
D = 256
NSEG = 16
NC = 2
NS = 16
NW = NC * NS
ROWS_PER_W = TOTAL // NW
CHUNK = 128
NCHUNK = ROWS_PER_W // CHUNK
LANES = 16
NVEC = D // LANES


def _sc_segsum_body(
    flat_hbm, cu_hbm, psum_hbm, cu_v, buf0, buf1, acc_v, sem0, sem1
):
  cid = lax.axis_index("c")
  sid = lax.axis_index("s")
  wid = sid * NC + cid
  base = wid * ROWS_PER_W

  pltpu.sync_copy(cu_hbm, cu_v)

  def cu_at(k):
    return cu_v[pl.ds(k, LANES)][0]

  zeros = jnp.zeros((LANES,), jnp.float32)

  @pl.loop(0, NSEG)
  def _zero(s):
    for j in range(NVEC):
      acc_v[s, pl.ds(j * LANES, LANES)] = zeros

  bufs = (buf0, buf1)
  sems = (sem0, sem1)

  pltpu.async_copy(flat_hbm.at[pl.ds(base, CHUNK)], bufs[0], sems[0])

  def compute(buf, row0):
    @pl.loop(0, NSEG)
    def _seg(k):
      lo = jnp.maximum(cu_at(k), row0) - row0
      hi = jnp.minimum(cu_at(k + 1), row0 + CHUNK) - row0

      @pl.when(hi > lo)
      def _():
        init = tuple(acc_v[k, pl.ds(j * LANES, LANES)] for j in range(NVEC))

        @pl.loop(lo, hi, init_carry=init)
        def acc(r, carry):
          return tuple(
              carry[j] + buf[r, pl.ds(j * LANES, LANES)]
              for j in range(NVEC)
          )

        for j in range(NVEC):
          acc_v[k, pl.ds(j * LANES, LANES)] = acc[j]

  @pl.loop(0, NCHUNK, step=2)
  def _pair(ch):
    for b in range(2):
      cur = ch + b

      @pl.when(cur + 1 < NCHUNK)
      def _():
        pltpu.async_copy(
            flat_hbm.at[pl.ds(base + (cur + 1) * CHUNK, CHUNK)],
            bufs[1 - b],
            sems[1 - b],
        )

      pltpu.make_async_copy(
          flat_hbm.at[pl.ds(0, CHUNK)], bufs[b], sems[b]
      ).wait()
      compute(bufs[b], base + cur * CHUNK)

  pltpu.sync_copy(acc_v, psum_hbm.at[wid])


@functools.partial(
    pl.kernel,
    out_type=jax.ShapeDtypeStruct((NW, NSEG, D), jnp.float32),
    mesh=plsc.VectorSubcoreMesh(core_axis_name="c", subcore_axis_name="s"),
    scratch_types=[
        pltpu.VMEM((2 * LANES,), jnp.int32),
        pltpu.VMEM((CHUNK, D), jnp.float32),
        pltpu.VMEM((CHUNK, D), jnp.float32),
        pltpu.VMEM((NSEG, D), jnp.float32),
        pltpu.SemaphoreType.DMA,
        pltpu.SemaphoreType.DMA,
    ],
)
def _sc_segsum(flat_hbm, cu_hbm, psum_hbm, cu_v, buf0, buf1, acc_v, s0, s1):
  _sc_segsum_body(flat_hbm, cu_hbm, psum_hbm, cu_v, buf0, buf1, acc_v, s0, s1)


def _mlp_body(cu_ref, psum_ref, w1_ref, b1_ref, w2_ref, b2_ref, out_ref):
  sums = jnp.sum(psum_ref[...], axis=0)
  scales = []
  for s in range(NSEG):
    cnt = (cu_ref[s + 1] - cu_ref[s]).astype(jnp.float32)
    scales.append(jnp.full((1, D), 1.0 / jnp.maximum(cnt, 1.0), jnp.float32))
  mean = sums * jnp.concatenate(scales, axis=0)
  h = jnp.maximum(
      jnp.dot(
          mean,
          w1_ref[...],
          preferred_element_type=jnp.float32,
          precision=lax.Precision.HIGHEST,
      )
      + b1_ref[...],
      0.0,
  )
  out_ref[...] = (
      jnp.dot(
          h,
          w2_ref[...],
          preferred_element_type=jnp.float32,
          precision=lax.Precision.HIGHEST,
      )
      + b2_ref[...]
  )


_mlp_call = pl.pallas_call(
    _mlp_body,
    out_shape=jax.ShapeDtypeStruct((NSEG, D), jnp.float32),
    in_specs=[
        pl.BlockSpec(memory_space=pltpu.SMEM),
        pl.BlockSpec(memory_space=pltpu.VMEM),
        pl.BlockSpec(memory_space=pltpu.VMEM),
        pl.BlockSpec(memory_space=pltpu.VMEM),
        pl.BlockSpec(memory_space=pltpu.VMEM),
        pl.BlockSpec(memory_space=pltpu.VMEM),
    ],
    out_specs=pl.BlockSpec(memory_space=pltpu.VMEM),
)


@jax.jit
def kernel(flat, cu_seqlens, W1, b1, W2, b2):
  cu_pad = jnp.concatenate(
      [cu_seqlens, jnp.full((2 * LANES - NSEG - 1,), TOTAL, jnp.int32)]
  )
  psum = _sc_segsum(flat, cu_pad)
  return _mlp_call(
      cu_seqlens, psum, W1, b1.reshape(1, -1), W2, b2.reshape(1, -1)
  )


# speedup vs baseline: 1.9690x; 1.0272x over previous
import functools

import jax
import jax.numpy as jnp
from jax import lax
from jax.experimental import pallas as pl
from jax.experimental.pallas import tpu as pltpu
from jax.experimental.pallas import tpu_sc as plsc

TOTAL = 32768
D = 256
NSEG = 16
NC = 2
NS = 16
NW = NC * NS
ROWS_PER_W = TOTAL // NW
CHUNK = 128
NCHUNK = ROWS_PER_W // CHUNK
LANES = 16
NVEC = D // LANES


def _sc_segsum_body(
    flat_hbm, cu_hbm, psum_hbm, cu_v, buf0, buf1, acc_v, sem0, sem1
):
  cid = lax.axis_index("c")
  sid = lax.axis_index("s")
  wid = sid * NC + cid
  base = wid * ROWS_PER_W

  pltpu.sync_copy(cu_hbm.at[pl.ds(0, LANES)], cu_v.at[pl.ds(0, LANES)])
  cu_v[pl.ds(LANES, LANES)] = jnp.full((LANES,), TOTAL, jnp.int32)

  def cu_at(k):
    return cu_v[pl.ds(k, LANES)][0]

  zeros = jnp.zeros((LANES,), jnp.float32)

  @pl.loop(0, NSEG)
  def _zero(s):
    for j in range(NVEC):
      acc_v[s, pl.ds(j * LANES, LANES)] = zeros

  bufs = (buf0, buf1)
  sems = (sem0, sem1)

  pltpu.async_copy(flat_hbm.at[pl.ds(base, CHUNK)], bufs[0], sems[0])

  def compute(buf, row0):
    @pl.loop(0, NSEG)
    def _seg(k):
      lo = jnp.maximum(cu_at(k), row0) - row0
      hi = jnp.minimum(cu_at(k + 1), row0 + CHUNK) - row0

      @pl.when(hi > lo)
      def _():
        init = tuple(acc_v[k, pl.ds(j * LANES, LANES)] for j in range(NVEC))

        @pl.loop(lo, hi, init_carry=init)
        def acc(r, carry):
          return tuple(
              carry[j] + buf[r, pl.ds(j * LANES, LANES)]
              for j in range(NVEC)
          )

        for j in range(NVEC):
          acc_v[k, pl.ds(j * LANES, LANES)] = acc[j]

  @pl.loop(0, NCHUNK, step=2)
  def _pair(ch):
    for b in range(2):
      cur = ch + b

      @pl.when(cur + 1 < NCHUNK)
      def _():
        pltpu.async_copy(
            flat_hbm.at[pl.ds(base + (cur + 1) * CHUNK, CHUNK)],
            bufs[1 - b],
            sems[1 - b],
        )

      pltpu.make_async_copy(
          flat_hbm.at[pl.ds(0, CHUNK)], bufs[b], sems[b]
      ).wait()
      compute(bufs[b], base + cur * CHUNK)

  pltpu.sync_copy(acc_v, psum_hbm.at[wid])


@functools.partial(
    pl.kernel,
    out_type=jax.ShapeDtypeStruct((NW, NSEG, D), jnp.float32),
    mesh=plsc.VectorSubcoreMesh(core_axis_name="c", subcore_axis_name="s"),
    scratch_types=[
        pltpu.VMEM((2 * LANES,), jnp.int32),
        pltpu.VMEM((CHUNK, D), jnp.float32),
        pltpu.VMEM((CHUNK, D), jnp.float32),
        pltpu.VMEM((NSEG, D), jnp.float32),
        pltpu.SemaphoreType.DMA,
        pltpu.SemaphoreType.DMA,
    ],
)
def _sc_segsum(flat_hbm, cu_hbm, psum_hbm, cu_v, buf0, buf1, acc_v, s0, s1):
  _sc_segsum_body(flat_hbm, cu_hbm, psum_hbm, cu_v, buf0, buf1, acc_v, s0, s1)


def _mlp_body(cu_ref, psum_ref, w1_ref, b1_ref, w2_ref, b2_ref, out_ref):
  sums = jnp.sum(psum_ref[...], axis=0)
  scales = []
  for s in range(NSEG):
    cnt = (cu_ref[s + 1] - cu_ref[s]).astype(jnp.float32)
    scales.append(jnp.full((1, D), 1.0 / jnp.maximum(cnt, 1.0), jnp.float32))
  mean = sums * jnp.concatenate(scales, axis=0)
  h = jnp.maximum(
      jnp.dot(
          mean,
          w1_ref[...],
          preferred_element_type=jnp.float32,
          precision=lax.Precision.HIGHEST,
      )
      + b1_ref[...],
      0.0,
  )
  out_ref[...] = (
      jnp.dot(
          h,
          w2_ref[...],
          preferred_element_type=jnp.float32,
          precision=lax.Precision.HIGHEST,
      )
      + b2_ref[...]
  )


_mlp_call = pl.pallas_call(
    _mlp_body,
    out_shape=jax.ShapeDtypeStruct((NSEG, D), jnp.float32),
    in_specs=[
        pl.BlockSpec(memory_space=pltpu.SMEM),
        pl.BlockSpec(memory_space=pltpu.VMEM),
        pl.BlockSpec(memory_space=pltpu.VMEM),
        pl.BlockSpec(memory_space=pltpu.VMEM),
        pl.BlockSpec(memory_space=pltpu.VMEM),
        pl.BlockSpec(memory_space=pltpu.VMEM),
    ],
    out_specs=pl.BlockSpec(memory_space=pltpu.VMEM),
)


@jax.jit
def kernel(flat, cu_seqlens, W1, b1, W2, b2):
  psum = _sc_segsum(flat, cu_seqlens)
  return _mlp_call(
      cu_seqlens, psum, W1, b1.reshape(1, -1), W2, b2.reshape(1, -1)
  )
